# baseline (speedup 1.0000x reference)
import functools

import jax
import jax.numpy as jnp
from jax import lax
from jax.experimental import pallas as pl
from jax.experimental.pallas import tpu as pltpu
from jax.experimental.pallas import tpu_sc as plsc

_NE = 1024
_D = 64
_B = 16
_L = 1024
_ROWS = _B * _L
_BLK = 1024
_GRID = _ROWS // _BLK
_NW = 32
_PER_W = _ROWS // _NW


def _vq_body(xt_ref, w_ref, wt_ref, enc_ref, idx_ref, loss_ref, perp_ref,
             counts_scr, sq_scr, b_scr):
    i = pl.program_id(0)

    @pl.when(i == 0)
    def _init():
        counts_scr[...] = jnp.zeros_like(counts_scr)
        sq_scr[0, 0] = 0.0
        b_scr[...] = jnp.sum(w_ref[...] * w_ref[...], axis=1)[None, :]

    xt = xt_ref[0]
    wt = wt_ref[...]

    m = jax.lax.dot_general(xt, wt, (((0,), (0,)), ((), ())),
                            preferred_element_type=jnp.float32)
    a = jnp.sum(xt * xt, axis=0)[:, None]
    b = b_scr[0]
    d = a + b[None, :] - 2.0 * m

    dmin = jnp.min(d, axis=1)
    iota = jax.lax.broadcasted_iota(jnp.int32, (1, _NE), 1).astype(jnp.float32)
    idx = jnp.min(jnp.where(d == dmin[:, None], iota, float(_NE)), axis=1)
    enc = (iota == idx[:, None]).astype(jnp.float32)

    enc_ref[...] = enc
    idx_ref[0, 0, :] = idx.astype(jnp.int32)

    counts_scr[...] += jnp.sum(enc, axis=0)[None, :]
    sq_scr[0, 0] += jnp.sum(dmin)

    @pl.when(i == _GRID - 1)
    def _fin():
        n_elems = float(_ROWS * _D)
        loss_ref[0, 0] = 1.25 * sq_scr[0, 0] / n_elems
        p = counts_scr[...] / float(_ROWS)
        ent = jnp.sum(p * jnp.log(p + 1e-10))
        perp_ref[0, 0] = jnp.exp(-ent)


_CHUNK = 128
_NCHUNK = _PER_W // _CHUNK


def _sc_gather(w_hbm, idx_hbm, out_hbm, idx_v, rows_v, sem):
    wid = lax.axis_index("s") * 2 + lax.axis_index("c")
    pltpu.sync_copy(idx_hbm.at[wid], idx_v)
    copies = []
    for j in range(_NCHUNK):
        copies.append(pltpu.async_copy(
            w_hbm.at[idx_v.at[j]],
            rows_v.at[pl.ds(j * _CHUNK, _CHUNK)],
            sem,
        ))
    for c in copies:
        c.wait()
    pltpu.sync_copy(rows_v, out_hbm.at[pl.ds(wid * _PER_W, _PER_W)])


def kernel(inputs, W):
    wt = W.T
    enc, idx3, loss, perp = pl.pallas_call(
        _vq_body,
        grid=(_GRID,),
        in_specs=[
            pl.BlockSpec((1, _D, _BLK), lambda i: (i, 0, 0)),
            pl.BlockSpec((_NE, _D), lambda i: (0, 0)),
            pl.BlockSpec((_D, _NE), lambda i: (0, 0)),
        ],
        out_specs=[
            pl.BlockSpec((_BLK, _NE), lambda i: (i, 0)),
            pl.BlockSpec((1, 1, _BLK), lambda i: (i, 0, 0)),
            pl.BlockSpec(memory_space=pltpu.SMEM),
            pl.BlockSpec(memory_space=pltpu.SMEM),
        ],
        out_shape=[
            jax.ShapeDtypeStruct((_ROWS, _NE), jnp.float32),
            jax.ShapeDtypeStruct((_GRID, 1, _BLK), jnp.int32),
            jax.ShapeDtypeStruct((1, 1), jnp.float32),
            jax.ShapeDtypeStruct((1, 1), jnp.float32),
        ],
        scratch_shapes=[
            pltpu.VMEM((1, _NE), jnp.float32),
            pltpu.SMEM((1, 1), jnp.float32),
            pltpu.VMEM((1, _NE), jnp.float32),
        ],
    )(inputs, W, wt)

    idx_chunked = idx3.reshape(_NW, _NCHUNK, _CHUNK)
    w_pad = jnp.pad(W, ((0, 0), (0, 128 - _D)))
    mesh = plsc.VectorSubcoreMesh(core_axis_name="c", subcore_axis_name="s")
    gather = functools.partial(
        pl.kernel,
        mesh=mesh,
        out_type=jax.ShapeDtypeStruct((_ROWS, 128), jnp.float32),
        scratch_types=[
            pltpu.VMEM((_NCHUNK, _CHUNK), jnp.int32),
            pltpu.VMEM((_PER_W, 128), jnp.float32),
            pltpu.SemaphoreType.DMA,
        ],
    )(_sc_gather)
    q_pad = gather(w_pad, idx_chunked)
    quantized_out = jnp.transpose(q_pad[:, :_D].reshape(_B, _L, _D), (0, 2, 1))
    return (loss[0, 0], quantized_out, perp[0, 0], enc)


# speedup vs baseline: 1.6839x; 1.6839x over previous
import jax
import jax.numpy as jnp
from jax.experimental import pallas as pl
from jax.experimental.pallas import tpu as pltpu

_NE = 1024
_D = 64
_B = 16
_L = 1024
_ROWS = _B * _L
_BLK = 1024
_GRID = _ROWS // _BLK


def _vq_body(xt_ref, w_ref, enc_ref, q_ref, loss_ref, perp_ref,
             counts_scr, sq_scr, b_scr, wt_scr):
    i = pl.program_id(0)

    @pl.when(i == 0)
    def _init():
        counts_scr[...] = jnp.zeros_like(counts_scr)
        sq_scr[0, 0] = 0.0
        b_scr[...] = jnp.sum(w_ref[...] * w_ref[...], axis=1)[None, :]
        wt_scr[...] = w_ref[...].T

    xt = xt_ref[0]
    w = w_ref[...]
    wt = wt_scr[...]

    m = jax.lax.dot_general(xt, wt, (((0,), (0,)), ((), ())),
                            preferred_element_type=jnp.float32)
    a = jnp.sum(xt * xt, axis=0)[:, None]
    b = b_scr[0]
    d = a + b[None, :] - 2.0 * m

    dmin = jnp.min(d, axis=1)
    iota = jax.lax.broadcasted_iota(jnp.int32, (1, _NE), 1).astype(jnp.float32)
    idx = jnp.min(jnp.where(d == dmin[:, None], iota, float(_NE)), axis=1)
    enc = (iota == idx[:, None]).astype(jnp.float32)

    enc_ref[...] = enc
    q_ref[0] = jax.lax.dot_general(w, enc, (((0,), (1,)), ((), ())),
                                   preferred_element_type=jnp.float32)

    counts_scr[...] += jnp.sum(enc, axis=0)[None, :]
    sq_scr[0, 0] += jnp.sum(dmin)

    @pl.when(i == _GRID - 1)
    def _fin():
        n_elems = float(_ROWS * _D)
        loss_ref[0, 0] = 1.25 * sq_scr[0, 0] / n_elems
        p = counts_scr[...] / float(_ROWS)
        ent = jnp.sum(p * jnp.log(p + 1e-10))
        perp_ref[0, 0] = jnp.exp(-ent)


def kernel(inputs, W):
    enc, q, loss, perp = pl.pallas_call(
        _vq_body,
        grid=(_GRID,),
        in_specs=[
            pl.BlockSpec((1, _D, _BLK), lambda i: (i, 0, 0)),
            pl.BlockSpec((_NE, _D), lambda i: (0, 0)),
        ],
        out_specs=[
            pl.BlockSpec((_BLK, _NE), lambda i: (i, 0)),
            pl.BlockSpec((1, _D, _BLK), lambda i: (i, 0, 0)),
            pl.BlockSpec(memory_space=pltpu.SMEM),
            pl.BlockSpec(memory_space=pltpu.SMEM),
        ],
        out_shape=[
            jax.ShapeDtypeStruct((_ROWS, _NE), jnp.float32),
            jax.ShapeDtypeStruct((_B, _D, _L), jnp.float32),
            jax.ShapeDtypeStruct((1, 1), jnp.float32),
            jax.ShapeDtypeStruct((1, 1), jnp.float32),
        ],
        scratch_shapes=[
            pltpu.VMEM((1, _NE), jnp.float32),
            pltpu.SMEM((1, 1), jnp.float32),
            pltpu.VMEM((1, _NE), jnp.float32),
            pltpu.VMEM((_D, _NE), jnp.float32),
        ],
    )(inputs, W)
    return (loss[0, 0], q, perp[0, 0], enc)


# speedup vs baseline: 1.8018x; 1.0700x over previous
import jax
import jax.numpy as jnp
from jax.experimental import pallas as pl
from jax.experimental.pallas import tpu as pltpu

_NE = 1024
_D = 64
_B = 16
_L = 1024
_ROWS = _B * _L
_BLK = 1024
_GRID = _ROWS // _BLK


def _vq_body(xt_ref, w_ref, enc_ref, q_ref, loss_ref, perp_ref,
             counts_scr, sq_scr, bcol_scr, icol_scr, wt_scr):
    i = pl.program_id(0)

    @pl.when(i == 0)
    def _init():
        counts_scr[...] = jnp.zeros_like(counts_scr)
        sq_scr[0, 0] = 0.0
        w0 = w_ref[...]
        wt_scr[...] = w0.T
        bcol_scr[...] = jnp.sum(w0 * w0, axis=1)[:, None]
        icol_scr[...] = jax.lax.broadcasted_iota(
            jnp.int32, (_NE, 1), 0).astype(jnp.float32)

    xt = xt_ref[0]
    w = w_ref[...]
    wt = wt_scr[...]

    mT = jax.lax.dot_general(wt, xt, (((0,), (0,)), ((), ())),
                             preferred_element_type=jnp.float32)
    a = jnp.sum(xt * xt, axis=0)[None, :]
    dT = a + bcol_scr[...] - 2.0 * mT

    dmin = jnp.min(dT, axis=0)
    idx = jnp.min(jnp.where(dT == dmin[None, :], icol_scr[...], float(_NE)),
                  axis=0)

    iota_row = jax.lax.broadcasted_iota(jnp.int32, (1, _NE), 1).astype(
        jnp.float32)
    enc = (iota_row == idx[:, None]).astype(jnp.float32)

    enc_ref[...] = enc
    q_ref[0] = jax.lax.dot_general(w, enc, (((0,), (1,)), ((), ())),
                                   preferred_element_type=jnp.float32)

    counts_scr[...] += jnp.sum(enc, axis=0)[None, :]
    sq_scr[0, 0] += jnp.sum(dmin)

    @pl.when(i == _GRID - 1)
    def _fin():
        n_elems = float(_ROWS * _D)
        loss_ref[0, 0] = 1.25 * sq_scr[0, 0] / n_elems
        p = counts_scr[...] / float(_ROWS)
        ent = jnp.sum(p * jnp.log(p + 1e-10))
        perp_ref[0, 0] = jnp.exp(-ent)


def kernel(inputs, W):
    enc, q, loss, perp = pl.pallas_call(
        _vq_body,
        grid=(_GRID,),
        in_specs=[
            pl.BlockSpec((1, _D, _BLK), lambda i: (i, 0, 0)),
            pl.BlockSpec((_NE, _D), lambda i: (0, 0)),
        ],
        out_specs=[
            pl.BlockSpec((_BLK, _NE), lambda i: (i, 0)),
            pl.BlockSpec((1, _D, _BLK), lambda i: (i, 0, 0)),
            pl.BlockSpec(memory_space=pltpu.SMEM),
            pl.BlockSpec(memory_space=pltpu.SMEM),
        ],
        out_shape=[
            jax.ShapeDtypeStruct((_ROWS, _NE), jnp.float32),
            jax.ShapeDtypeStruct((_B, _D, _L), jnp.float32),
            jax.ShapeDtypeStruct((1, 1), jnp.float32),
            jax.ShapeDtypeStruct((1, 1), jnp.float32),
        ],
        scratch_shapes=[
            pltpu.VMEM((1, _NE), jnp.float32),
            pltpu.SMEM((1, 1), jnp.float32),
            pltpu.VMEM((_NE, 1), jnp.float32),
            pltpu.VMEM((_NE, 1), jnp.float32),
            pltpu.VMEM((_D, _NE), jnp.float32),
        ],
    )(inputs, W)
    return (loss[0, 0], q, perp[0, 0], enc)


# speedup vs baseline: 1.8190x; 1.0095x over previous
import jax
import jax.numpy as jnp
from jax.experimental import pallas as pl
from jax.experimental.pallas import tpu as pltpu

_NE = 1024
_D = 64
_B = 16
_L = 1024
_ROWS = _B * _L
_BLK = 1024
_PER_STEP = 2
_GRID = _B // _PER_STEP


def _vq_body(xt_ref, w_ref, enc_ref, q_ref, loss_ref, perp_ref,
             counts_scr, sq_scr, bcol_scr, icol_scr, wt_scr):
    i = pl.program_id(0)

    @pl.when(i == 0)
    def _init():
        counts_scr[...] = jnp.zeros_like(counts_scr)
        sq_scr[0, 0] = 0.0
        w0 = w_ref[...]
        wt_scr[...] = w0.T
        bcol_scr[...] = jnp.sum(w0 * w0, axis=1)[:, None]
        icol_scr[...] = jax.lax.broadcasted_iota(
            jnp.int32, (_NE, 1), 0).astype(jnp.float32)

    w = w_ref[...]
    wt = wt_scr[...]
    bcol = bcol_scr[...]
    icol = icol_scr[...]
    iota_row = jax.lax.broadcasted_iota(jnp.int32, (1, _NE), 1).astype(
        jnp.float32)

    csum = jnp.zeros((1, _NE), jnp.float32)
    ssum = 0.0
    for k in range(_PER_STEP):
        xt = xt_ref[k]
        mT = jax.lax.dot_general(wt, xt, (((0,), (0,)), ((), ())),
                                 preferred_element_type=jnp.float32)
        a = jnp.sum(xt * xt, axis=0)[None, :]
        dT = a + bcol - 2.0 * mT

        dmin = jnp.min(dT, axis=0)
        idx = jnp.min(jnp.where(dT == dmin[None, :], icol, float(_NE)),
                      axis=0)
        enc = (iota_row == idx[:, None]).astype(jnp.float32)

        enc_ref[pl.ds(k * _BLK, _BLK), :] = enc
        q_ref[k] = jax.lax.dot_general(w, enc, (((0,), (1,)), ((), ())),
                                       preferred_element_type=jnp.float32)
        csum = csum + jnp.sum(enc, axis=0)[None, :]
        ssum = ssum + jnp.sum(dmin)

    counts_scr[...] += csum
    sq_scr[0, 0] += ssum

    @pl.when(i == _GRID - 1)
    def _fin():
        n_elems = float(_ROWS * _D)
        loss_ref[0, 0] = 1.25 * sq_scr[0, 0] / n_elems
        p = counts_scr[...] / float(_ROWS)
        ent = jnp.sum(p * jnp.log(p + 1e-10))
        perp_ref[0, 0] = jnp.exp(-ent)


def kernel(inputs, W):
    enc, q, loss, perp = pl.pallas_call(
        _vq_body,
        grid=(_GRID,),
        in_specs=[
            pl.BlockSpec((_PER_STEP, _D, _BLK), lambda i: (i, 0, 0)),
            pl.BlockSpec((_NE, _D), lambda i: (0, 0)),
        ],
        out_specs=[
            pl.BlockSpec((_PER_STEP * _BLK, _NE), lambda i: (i, 0)),
            pl.BlockSpec((_PER_STEP, _D, _BLK), lambda i: (i, 0, 0)),
            pl.BlockSpec(memory_space=pltpu.SMEM),
            pl.BlockSpec(memory_space=pltpu.SMEM),
        ],
        out_shape=[
            jax.ShapeDtypeStruct((_ROWS, _NE), jnp.float32),
            jax.ShapeDtypeStruct((_B, _D, _L), jnp.float32),
            jax.ShapeDtypeStruct((1, 1), jnp.float32),
            jax.ShapeDtypeStruct((1, 1), jnp.float32),
        ],
        scratch_shapes=[
            pltpu.VMEM((1, _NE), jnp.float32),
            pltpu.SMEM((1, 1), jnp.float32),
            pltpu.VMEM((_NE, 1), jnp.float32),
            pltpu.VMEM((_NE, 1), jnp.float32),
            pltpu.VMEM((_D, _NE), jnp.float32),
        ],
    )(inputs, W)
    return (loss[0, 0], q, perp[0, 0], enc)


# speedup vs baseline: 1.9166x; 1.0537x over previous
import jax
import jax.numpy as jnp
from jax.experimental import pallas as pl
from jax.experimental.pallas import tpu as pltpu

_NE = 1024
_D = 64
_B = 16
_L = 1024
_ROWS = _B * _L
_BLK = 1024
_PER_STEP = 2
_GRID = _B // _PER_STEP


def _vq_body(xt_ref, w_ref, enc_ref, q_ref, loss_ref, perp_ref,
             counts_scr, sq_scr, bcol_scr, icol_scr, wt_scr):
    i = pl.program_id(0)

    @pl.when(i == 0)
    def _init():
        counts_scr[...] = jnp.zeros_like(counts_scr)
        sq_scr[0, 0] = 0.0
        w0 = w_ref[...]
        wt_scr[...] = w0.T
        bcol_scr[...] = jnp.sum(w0 * w0, axis=1)[:, None]
        icol_scr[...] = jax.lax.broadcasted_iota(
            jnp.int32, (_NE, 1), 0).astype(jnp.float32)

    w = w_ref[...]
    wt = wt_scr[...]
    bcol = bcol_scr[...]
    icol = icol_scr[...]
    iota_row = jax.lax.broadcasted_iota(jnp.int32, (1, _NE), 1).astype(
        jnp.float32)

    xts = [xt_ref[k] for k in range(_PER_STEP)]
    mTs = [jax.lax.dot_general(wt, xt, (((0,), (0,)), ((), ())),
                               preferred_element_type=jnp.float32)
           for xt in xts]

    csum = jnp.zeros((1, _NE), jnp.float32)
    ssum = 0.0
    for k in range(_PER_STEP):
        xt = xts[k]
        mT = mTs[k]
        a = jnp.sum(xt * xt, axis=0)[None, :]
        dT = a + bcol - 2.0 * mT

        dmin = jnp.min(dT, axis=0)
        idx = jnp.min(jnp.where(dT == dmin[None, :], icol, float(_NE)),
                      axis=0)
        enc = (iota_row == idx[:, None]).astype(jnp.float32)

        enc_ref[pl.ds(k * _BLK, _BLK), :] = enc
        q_ref[k] = jax.lax.dot_general(w, enc, (((0,), (1,)), ((), ())),
                                       preferred_element_type=jnp.float32)
        csum = csum + jnp.sum(enc, axis=0)[None, :]
        ssum = ssum + jnp.sum(dmin)

    counts_scr[...] += csum
    sq_scr[0, 0] += ssum

    @pl.when(i == _GRID - 1)
    def _fin():
        n_elems = float(_ROWS * _D)
        loss_ref[0, 0] = 1.25 * sq_scr[0, 0] / n_elems
        p = counts_scr[...] / float(_ROWS)
        ent = jnp.sum(p * jnp.log(p + 1e-10))
        perp_ref[0, 0] = jnp.exp(-ent)


def kernel(inputs, W):
    enc, q, loss, perp = pl.pallas_call(
        _vq_body,
        grid=(_GRID,),
        in_specs=[
            pl.BlockSpec((_PER_STEP, _D, _BLK), lambda i: (i, 0, 0)),
            pl.BlockSpec((_NE, _D), lambda i: (0, 0)),
        ],
        out_specs=[
            pl.BlockSpec((_PER_STEP * _BLK, _NE), lambda i: (i, 0)),
            pl.BlockSpec((_PER_STEP, _D, _BLK), lambda i: (i, 0, 0)),
            pl.BlockSpec(memory_space=pltpu.SMEM),
            pl.BlockSpec(memory_space=pltpu.SMEM),
        ],
        out_shape=[
            jax.ShapeDtypeStruct((_ROWS, _NE), jnp.float32),
            jax.ShapeDtypeStruct((_B, _D, _L), jnp.float32),
            jax.ShapeDtypeStruct((1, 1), jnp.float32),
            jax.ShapeDtypeStruct((1, 1), jnp.float32),
        ],
        scratch_shapes=[
            pltpu.VMEM((1, _NE), jnp.float32),
            pltpu.SMEM((1, 1), jnp.float32),
            pltpu.VMEM((_NE, 1), jnp.float32),
            pltpu.VMEM((_NE, 1), jnp.float32),
            pltpu.VMEM((_D, _NE), jnp.float32),
        ],
    )(inputs, W)
    return (loss[0, 0], q, perp[0, 0], enc)


# speedup vs baseline: 2.0718x; 1.0810x over previous
import jax
import jax.numpy as jnp
from jax.experimental import pallas as pl
from jax.experimental.pallas import tpu as pltpu

_NE = 1024
_D = 64
_B = 16
_L = 1024
_ROWS = _B * _L
_BLK = 1024
_PER_STEP = 2
_GRID = _B // _PER_STEP


def _vq_body(xt_ref, w_ref, enc_ref, q_ref, loss_ref, perp_ref,
             counts_scr, sq_scr, bcol_scr, wt_scr):
    i = pl.program_id(0)

    @pl.when(i == 0)
    def _init():
        counts_scr[...] = jnp.zeros_like(counts_scr)
        sq_scr[0, 0] = 0.0
        w0 = w_ref[...]
        wt_scr[...] = w0.T
        bcol_scr[...] = jnp.sum(w0 * w0, axis=1)[:, None]

    w = w_ref[...]
    wt = wt_scr[...]
    bcol = bcol_scr[...]
    iota_row = jax.lax.broadcasted_iota(jnp.int32, (1, _NE), 1)

    xts = [xt_ref[k] for k in range(_PER_STEP)]
    mTs = [jax.lax.dot_general(wt, xt, (((0,), (0,)), ((), ())),
                               preferred_element_type=jnp.float32)
           for xt in xts]

    csum = jnp.zeros((1, _NE), jnp.float32)
    ssum = 0.0
    for k in range(_PER_STEP):
        xt = xts[k]
        mT = mTs[k]
        a = jnp.sum(xt * xt, axis=0)[None, :]
        dT = a + bcol - 2.0 * mT

        idx = jnp.argmin(dT, axis=0)
        enc = (iota_row == idx[:, None]).astype(jnp.float32)

        enc_ref[pl.ds(k * _BLK, _BLK), :] = enc
        qt = jax.lax.dot_general(w, enc, (((0,), (1,)), ((), ())),
                                 preferred_element_type=jnp.float32)
        q_ref[k] = qt
        csum = csum + jnp.sum(enc, axis=0)[None, :]
        diff = qt - xt
        ssum = ssum + jnp.sum(diff * diff)

    counts_scr[...] += csum
    sq_scr[0, 0] += ssum

    @pl.when(i == _GRID - 1)
    def _fin():
        n_elems = float(_ROWS * _D)
        loss_ref[0, 0] = 1.25 * sq_scr[0, 0] / n_elems
        p = counts_scr[...] / float(_ROWS)
        ent = jnp.sum(p * jnp.log(p + 1e-10))
        perp_ref[0, 0] = jnp.exp(-ent)


def kernel(inputs, W):
    enc, q, loss, perp = pl.pallas_call(
        _vq_body,
        grid=(_GRID,),
        in_specs=[
            pl.BlockSpec((_PER_STEP, _D, _BLK), lambda i: (i, 0, 0)),
            pl.BlockSpec((_NE, _D), lambda i: (0, 0)),
        ],
        out_specs=[
            pl.BlockSpec((_PER_STEP * _BLK, _NE), lambda i: (i, 0)),
            pl.BlockSpec((_PER_STEP, _D, _BLK), lambda i: (i, 0, 0)),
            pl.BlockSpec(memory_space=pltpu.SMEM),
            pl.BlockSpec(memory_space=pltpu.SMEM),
        ],
        out_shape=[
            jax.ShapeDtypeStruct((_ROWS, _NE), jnp.float32),
            jax.ShapeDtypeStruct((_B, _D, _L), jnp.float32),
            jax.ShapeDtypeStruct((1, 1), jnp.float32),
            jax.ShapeDtypeStruct((1, 1), jnp.float32),
        ],
        scratch_shapes=[
            pltpu.VMEM((1, _NE), jnp.float32),
            pltpu.SMEM((1, 1), jnp.float32),
            pltpu.VMEM((_NE, 1), jnp.float32),
            pltpu.VMEM((_D, _NE), jnp.float32),
        ],
    )(inputs, W)
    return (loss[0, 0], q, perp[0, 0], enc)
